# Initial kernel scaffold; baseline (speedup 1.0000x reference)
import jax
import jax.numpy as jnp
from jax.experimental import pallas as pl


def kernel(x, edge_index, edge_attr, batchs, x_apo, edge_index_apo, edge_attr_apo, params):
    raise NotImplementedError("write your pallas kernel here")


# speedup vs baseline: 1.8729x; 1.8729x over previous
import math

import jax
import jax.numpy as jnp
from jax import lax
from jax.experimental import pallas as pl
from jax.experimental.pallas import tpu as pltpu
from jax.experimental.pallas import tpu_sc as plsc

N = 10000
E = 160000
D = 128
NUM_GRAPHS = 64

NC = 2
NS = 16
NW = NC * NS
L = 16

N_PAD = 10240
C = 128
CH = 80
E_PAD = NS * CH * C
NH = N_PAD // 2
OH = NH + 128
INV_SQRT_D = 1.0 / math.sqrt(float(D))


def _edge_body(q_hbm, k_hbm, v_hbm, src_hbm, dst_hbm, e_hbm,
               po_hbm, den_hbm,
               q_rows, k_rows, v_rows, e_rows,
               src_v, dst_v, dst_loc, den_tile, out_sh, sem):
    cid = lax.axis_index("c")
    sid = lax.axis_index("s")
    rows = OH // NS
    zero16 = jnp.zeros((L,), jnp.float32)

    def zden(i, carry):
        den_tile[pl.ds(i * L, L)] = zero16
        return carry

    lax.fori_loop(0, N_PAD // L, zden, None)

    def zrows(i, carry):
        for j in range(D // L):
            v_rows[i, pl.ds(j * L, L)] = zero16
        return carry

    lax.fori_loop(0, C, zrows, None)
    zbase = sid * rows
    for t in range(rows // C):
        pltpu.sync_copy(v_rows, out_sh.at[pl.ds(zbase + t * C, C)])
    rem = rows % C
    if rem:
        pltpu.sync_copy(v_rows.at[pl.ds(0, rem)],
                        out_sh.at[pl.ds(zbase + (rows // C) * C, rem)])
    plsc.subcore_barrier()
    half_base = cid * NH

    lane = lax.iota(jnp.int32, L)

    def chunk(c, carry):
        pltpu.sync_copy(src_hbm.at[sid, c], src_v)
        pltpu.sync_copy(dst_hbm.at[sid, c], dst_v)
        pltpu.sync_copy(e_hbm.at[sid, c], e_rows)
        pltpu.async_copy(q_hbm.at[dst_v], q_rows, sem).wait()
        pltpu.async_copy(k_hbm.at[src_v], k_rows, sem).wait()
        pltpu.async_copy(v_hbm.at[src_v], v_rows, sem).wait()

        def dot16(b, inner):
            base = b * L
            alpha16 = jnp.zeros((L,), jnp.float32)
            for l in range(L):
                i = base + l
                acc = jnp.zeros((L,), jnp.float32)
                for j in range(D // L):
                    sl = pl.ds(j * L, L)
                    acc = acc + q_rows[i, sl] * (k_rows[i, sl] + e_rows[i, sl])
                alpha16 = jnp.where(lane == l, jnp.sum(acc), alpha16)
            ex16 = jnp.exp(alpha16 * INV_SQRT_D)
            d16 = dst_v[pl.ds(base, L)]

            @pl.when(cid == 0)
            def _():
                plsc.addupdate_scatter(den_tile, [d16], ex16)

            dl = d16 - half_base
            dl = jnp.where((dl >= 0) & (dl < NH), dl, NH)
            dst_loc[pl.ds(base, L)] = dl
            for l in range(L):
                i = base + l
                ex = ex16[l]
                for j in range(D // L):
                    sl = pl.ds(j * L, L)
                    v_rows[i, sl] = (v_rows[i, sl] + e_rows[i, sl]) * ex
            return inner

        lax.fori_loop(0, C // L, dot16, None)

        pltpu.sync_copy(v_rows, out_sh.at[dst_loc], add=True)
        return carry

    lax.fori_loop(0, CH, chunk, None)

    @pl.when(cid == 0)
    def _():
        pltpu.sync_copy(den_tile, den_hbm.at[pl.ds(sid * N_PAD, N_PAD)])

    plsc.subcore_barrier()

    pltpu.sync_copy(out_sh.at[pl.ds(sid * rows, rows)],
                    po_hbm.at[pl.ds(cid * OH + sid * rows, rows)])


_edge_kernel = pl.kernel(
    _edge_body,
    out_type=(
        jax.ShapeDtypeStruct((NC * OH, D), jnp.float32),
        jax.ShapeDtypeStruct((NS * N_PAD,), jnp.float32),
    ),
    mesh=plsc.VectorSubcoreMesh(core_axis_name="c", subcore_axis_name="s",
                                num_cores=NC, num_subcores=NS),
    compiler_params=pltpu.CompilerParams(needs_layout_passes=False),
    scratch_types=[
        pltpu.VMEM((C, D), jnp.float32),
        pltpu.VMEM((C, D), jnp.float32),
        pltpu.VMEM((C, D), jnp.float32),
        pltpu.VMEM((C, D), jnp.float32),
        pltpu.VMEM((C,), jnp.int32),
        pltpu.VMEM((C,), jnp.int32),
        pltpu.VMEM((C,), jnp.int32),
        pltpu.VMEM((N_PAD,), jnp.float32),
        pltpu.VMEM_SHARED((OH, D), jnp.float32),
        pltpu.SemaphoreType.DMA,
    ],
)


def _proj_body(x_ref, w_ref, b_ref, o_ref):
    o_ref[...] = (jnp.dot(x_ref[...], w_ref[...],
                          preferred_element_type=jnp.float32) + b_ref[...])


def _proj_one(x_pad, w, b):
    B = 2048
    grid = N_PAD // B
    return pl.pallas_call(
        _proj_body,
        grid=(grid,),
        in_specs=[pl.BlockSpec((B, D), lambda i: (i, 0)),
                  pl.BlockSpec((D, D), lambda i: (0, 0)),
                  pl.BlockSpec((1, D), lambda i: (0, 0))],
        out_specs=pl.BlockSpec((B, D), lambda i: (i, 0)),
        out_shape=jax.ShapeDtypeStruct((N_PAD, D), jnp.float32),
    )(x_pad, w, b.reshape(1, D))


def _eproj_body(ea_ref, we_ref, e_ref):
    e_ref[...] = jnp.dot(ea_ref[...], we_ref[...],
                         preferred_element_type=jnp.float32)


def _eproj(ea_pad, we16):
    B = 2048
    grid = E_PAD // B
    return pl.pallas_call(
        _eproj_body,
        grid=(grid,),
        in_specs=[pl.BlockSpec((B, 16), lambda i: (i, 0)),
                  pl.BlockSpec((16, D), lambda i: (0, 0))],
        out_specs=pl.BlockSpec((B, D), lambda i: (i, 0)),
        out_shape=jax.ShapeDtypeStruct((E_PAD, D), jnp.float32),
    )(ea_pad, we16)


def _epilogue_body(po_ref, den_ref, s_ref, h_ref):
    den = jnp.sum(den_ref[...], axis=0)[:, None]
    h_ref[...] = jnp.maximum(po_ref[...] / (den + 1e-16) + s_ref[...], 0.0)


def _epilogue(po, den, s_pre):
    B = 1280
    grid = N_PAD // B
    return pl.pallas_call(
        _epilogue_body,
        grid=(grid,),
        in_specs=[pl.BlockSpec((B, D), lambda i: (i, 0)),
                  pl.BlockSpec((NS, B), lambda i: (0, i)),
                  pl.BlockSpec((B, D), lambda i: (i, 0))],
        out_specs=pl.BlockSpec((B, D), lambda i: (i, 0)),
        out_shape=jax.ShapeDtypeStruct((N_PAD, D), jnp.float32),
    )(po, den, s_pre)


def _pool_body(b_ref, h1_ref, h2_ref, s1_ref, s2_ref, c_ref):
    @pl.when(pl.program_id(0) == 0)
    def _():
        s1_ref[...] = jnp.zeros_like(s1_ref)
        s2_ref[...] = jnp.zeros_like(s2_ref)
        c_ref[...] = jnp.zeros_like(c_ref)

    b = b_ref[0, 0, :]
    oh = (b[None, :] == lax.broadcasted_iota(jnp.int32, (NUM_GRAPHS, b.shape[0]), 0)
          ).astype(jnp.float32)
    s1_ref[...] += jnp.dot(oh, h1_ref[...], preferred_element_type=jnp.float32)
    s2_ref[...] += jnp.dot(oh, h2_ref[...], preferred_element_type=jnp.float32)
    c_ref[...] += jnp.sum(oh, axis=1)[:, None]


def _pool(batchs, h1, h2):
    B = 1000
    grid = N // B
    b3 = batchs.reshape(grid, 1, B)
    out = lambda: pl.BlockSpec((NUM_GRAPHS, D), lambda i: (0, 0))
    return pl.pallas_call(
        _pool_body,
        grid=(grid,),
        in_specs=[pl.BlockSpec((1, 1, B), lambda i: (i, 0, 0)),
                  pl.BlockSpec((B, D), lambda i: (i, 0)),
                  pl.BlockSpec((B, D), lambda i: (i, 0))],
        out_specs=[out(), out(), out()],
        out_shape=[jax.ShapeDtypeStruct((NUM_GRAPHS, D), jnp.float32)] * 3,
    )(b3, h1, h2)


def _head_body(s1_ref, s2_ref, c_ref, w1, b1, w2, b2, w3, b3, o_ref):
    c = jnp.maximum(c_ref[...], 1.0)
    xc = jnp.concatenate([s1_ref[...] / c, s2_ref[...] / c], axis=1)
    xc = jnp.maximum(jnp.dot(xc, w1[...], preferred_element_type=jnp.float32)
                     + b1[...], 0.0)
    xc = jnp.maximum(jnp.dot(xc, w2[...], preferred_element_type=jnp.float32)
                     + b2[...], 0.0)
    xc = jnp.dot(xc, w3[...], preferred_element_type=jnp.float32) + b3[...]
    m = jnp.max(xc, axis=1, keepdims=True)
    e = jnp.exp(xc - m)
    o_ref[...] = e / jnp.sum(e, axis=1, keepdims=True)


def _head(s1, s2, cnt, params):
    return pl.pallas_call(
        _head_body,
        out_shape=jax.ShapeDtypeStruct((NUM_GRAPHS, 2), jnp.float32),
    )(s1, s2, cnt,
      params["linl_W"], params["linl_b"].reshape(1, 2 * D),
      params["linl2_W"], params["linl2_b"].reshape(1, D),
      params["fc_W"], params["fc_b"].reshape(1, 2))


def _prep_edges(edge_index, edge_attr):
    pad = jnp.full((E_PAD - E,), N, jnp.int32)
    src = jnp.concatenate([edge_index[0], pad]).reshape(NS, CH, C)
    dst = jnp.concatenate([edge_index[1], pad]).reshape(NS, CH, C)
    ed = edge_attr.shape[1]
    ea = jnp.pad(edge_attr, ((0, E_PAD - E), (0, 16 - ed)))
    return src, dst, ea


def _padw(w):
    return jnp.pad(w, ((0, D - w.shape[0]), (0, D - w.shape[1])))


def _stack_params(plist, names, padder):
    return {n: jnp.stack([padder(p[n]) for p in plist]) for n in names}


def kernel(x, edge_index, edge_attr, batchs, x_apo, edge_index_apo,
           edge_attr_apo, params):
    src1, dst1, ea1 = _prep_edges(edge_index, edge_attr)
    src2, dst2, ea2 = _prep_edges(edge_index_apo, edge_attr_apo)

    x1 = jnp.pad(x, ((0, N_PAD - N), (0, D - x.shape[1])))
    x2 = jnp.pad(x_apo, ((0, N_PAD - N), (0, D - x_apo.shape[1])))

    def layer(h_in, src, dst, ea, p):
        q = _proj_one(h_in, _padw(p["Wq"]), p["bq"])
        k = _proj_one(h_in, _padw(p["Wk"]), p["bk"])
        v = _proj_one(h_in, _padw(p["Wv"]), p["bv"])
        s_pre = _proj_one(h_in, _padw(p["Ws"]), p["bs"])
        we16 = jnp.pad(p["We"], ((0, 16 - p["We"].shape[0]), (0, 0)))
        e = _eproj(ea, we16).reshape(NS, CH, C, D)
        po, den = _edge_kernel(q, k, v, src, dst, e)
        po2 = po.reshape(NC, OH, D)[:, :NH, :].reshape(N_PAD, D)
        return _epilogue(po2, den.reshape(NS, N_PAD), s_pre)

    h = layer(x1, src1, dst1, ea1, params["conv1"])
    for p in params["convs"]:
        h = layer(h, src1, dst1, ea1, p)
    h2 = layer(x2, src2, dst2, ea2, params["conv2"])
    for p in params["convs2"]:
        h2 = layer(h2, src2, dst2, ea2, p)

    s1, s2, cnt = _pool(batchs, h[:N], h2[:N])
    return _head(s1, s2, cnt, params)


# speedup vs baseline: 2.4522x; 1.3093x over previous
import math

import jax
import jax.numpy as jnp
from jax import lax
from jax.experimental import pallas as pl
from jax.experimental.pallas import tpu as pltpu
from jax.experimental.pallas import tpu_sc as plsc

N = 10000
E = 160000
D = 128
NUM_GRAPHS = 64

NC = 2
NS = 16
NW = NC * NS
L = 16

N_PAD = 10240
C = 128
CH = 80
E_PAD = NS * CH * C
NH = N_PAD // 2
OH = NH + 128
INV_SQRT_D = 1.0 / math.sqrt(float(D))


def _edge_body(q_hbm, k_hbm, v_hbm, src_hbm, dst_hbm, e_hbm,
               po_hbm, den_hbm,
               q_rows, k_rows, v_rows, e_rows,
               src_v, dst_v, dst_loc, den_tile, out_sh, sem, sem2):
    cid = lax.axis_index("c")
    sid = lax.axis_index("s")
    rows = OH // NS
    zero16 = jnp.zeros((L,), jnp.float32)

    def zden(i, carry):
        den_tile[pl.ds(i * L, L)] = zero16
        return carry

    lax.fori_loop(0, N_PAD // L, zden, None)

    def zrows(i, carry):
        for j in range(D // L):
            v_rows[i, pl.ds(j * L, L)] = zero16
        return carry

    lax.fori_loop(0, C, zrows, None)
    zbase = sid * rows
    for t in range(rows // C):
        pltpu.sync_copy(v_rows, out_sh.at[pl.ds(zbase + t * C, C)])
    rem = rows % C
    if rem:
        pltpu.sync_copy(v_rows.at[pl.ds(0, rem)],
                        out_sh.at[pl.ds(zbase + (rows // C) * C, rem)])
    plsc.subcore_barrier()
    half_base = cid * NH

    lane = lax.iota(jnp.int32, L)

    def chunk(c, carry):
        d_src = pltpu.async_copy(src_hbm.at[sid, c], src_v, sem2)
        d_dst = pltpu.async_copy(dst_hbm.at[sid, c], dst_v, sem2)
        d_e = pltpu.async_copy(e_hbm.at[sid, c], e_rows, sem2)
        d_src.wait()
        d_dst.wait()
        d_q = pltpu.async_copy(q_hbm.at[dst_v], q_rows, sem)
        d_k = pltpu.async_copy(k_hbm.at[src_v], k_rows, sem)
        d_v = pltpu.async_copy(v_hbm.at[src_v], v_rows, sem)
        d_e.wait()
        d_q.wait()
        d_k.wait()
        d_v.wait()

        def dot16(b, inner):
            base = b * L
            alpha16 = jnp.zeros((L,), jnp.float32)
            for l in range(L):
                i = base + l
                acc = jnp.zeros((L,), jnp.float32)
                for j in range(D // L):
                    sl = pl.ds(j * L, L)
                    acc = acc + q_rows[i, sl] * (k_rows[i, sl] + e_rows[i, sl])
                alpha16 = jnp.where(lane == l, jnp.sum(acc), alpha16)
            ex16 = jnp.exp(alpha16 * INV_SQRT_D)
            d16 = dst_v[pl.ds(base, L)]

            @pl.when(cid == 0)
            def _():
                plsc.addupdate_scatter(den_tile, [d16], ex16)

            dl = d16 - half_base
            dl = jnp.where((dl >= 0) & (dl < NH), dl, NH)
            dst_loc[pl.ds(base, L)] = dl
            for l in range(L):
                i = base + l
                ex = ex16[l]
                for j in range(D // L):
                    sl = pl.ds(j * L, L)
                    v_rows[i, sl] = (v_rows[i, sl] + e_rows[i, sl]) * ex
            return inner

        lax.fori_loop(0, C // L, dot16, None)

        pltpu.sync_copy(v_rows, out_sh.at[dst_loc], add=True)
        return carry

    lax.fori_loop(0, CH, chunk, None)

    @pl.when(cid == 0)
    def _():
        pltpu.sync_copy(den_tile, den_hbm.at[pl.ds(sid * N_PAD, N_PAD)])

    plsc.subcore_barrier()

    pltpu.sync_copy(out_sh.at[pl.ds(sid * rows, rows)],
                    po_hbm.at[pl.ds(cid * OH + sid * rows, rows)])


_edge_kernel = pl.kernel(
    _edge_body,
    out_type=(
        jax.ShapeDtypeStruct((NC * OH, D), jnp.float32),
        jax.ShapeDtypeStruct((NS * N_PAD,), jnp.float32),
    ),
    mesh=plsc.VectorSubcoreMesh(core_axis_name="c", subcore_axis_name="s",
                                num_cores=NC, num_subcores=NS),
    compiler_params=pltpu.CompilerParams(needs_layout_passes=False),
    scratch_types=[
        pltpu.VMEM((C, D), jnp.float32),
        pltpu.VMEM((C, D), jnp.float32),
        pltpu.VMEM((C, D), jnp.float32),
        pltpu.VMEM((C, D), jnp.float32),
        pltpu.VMEM((C,), jnp.int32),
        pltpu.VMEM((C,), jnp.int32),
        pltpu.VMEM((C,), jnp.int32),
        pltpu.VMEM((N_PAD,), jnp.float32),
        pltpu.VMEM_SHARED((OH, D), jnp.float32),
        pltpu.SemaphoreType.DMA,
        pltpu.SemaphoreType.DMA,
    ],
)


def _proj_body(x_ref, w_ref, b_ref, o_ref):
    o_ref[...] = (jnp.dot(x_ref[...], w_ref[...],
                          preferred_element_type=jnp.float32) + b_ref[...])


def _proj_one(x_pad, w, b):
    B = 2048
    grid = N_PAD // B
    return pl.pallas_call(
        _proj_body,
        grid=(grid,),
        in_specs=[pl.BlockSpec((B, D), lambda i: (i, 0)),
                  pl.BlockSpec((D, D), lambda i: (0, 0)),
                  pl.BlockSpec((1, D), lambda i: (0, 0))],
        out_specs=pl.BlockSpec((B, D), lambda i: (i, 0)),
        out_shape=jax.ShapeDtypeStruct((N_PAD, D), jnp.float32),
    )(x_pad, w, b.reshape(1, D))


def _eproj_body(ea_ref, we_ref, e_ref):
    e_ref[...] = jnp.dot(ea_ref[...], we_ref[...],
                         preferred_element_type=jnp.float32)


def _eproj(ea_pad, we16):
    B = 2048
    grid = E_PAD // B
    return pl.pallas_call(
        _eproj_body,
        grid=(grid,),
        in_specs=[pl.BlockSpec((B, 16), lambda i: (i, 0)),
                  pl.BlockSpec((16, D), lambda i: (0, 0))],
        out_specs=pl.BlockSpec((B, D), lambda i: (i, 0)),
        out_shape=jax.ShapeDtypeStruct((E_PAD, D), jnp.float32),
    )(ea_pad, we16)


def _epilogue_body(po_ref, den_ref, s_ref, h_ref):
    den = jnp.sum(den_ref[...], axis=0)[:, None]
    h_ref[...] = jnp.maximum(po_ref[...] / (den + 1e-16) + s_ref[...], 0.0)


def _epilogue(po, den, s_pre):
    B = 1280
    grid = N_PAD // B
    return pl.pallas_call(
        _epilogue_body,
        grid=(grid,),
        in_specs=[pl.BlockSpec((B, D), lambda i: (i, 0)),
                  pl.BlockSpec((NS, B), lambda i: (0, i)),
                  pl.BlockSpec((B, D), lambda i: (i, 0))],
        out_specs=pl.BlockSpec((B, D), lambda i: (i, 0)),
        out_shape=jax.ShapeDtypeStruct((N_PAD, D), jnp.float32),
    )(po, den, s_pre)


def _pool_body(b_ref, h1_ref, h2_ref, s1_ref, s2_ref, c_ref):
    @pl.when(pl.program_id(0) == 0)
    def _():
        s1_ref[...] = jnp.zeros_like(s1_ref)
        s2_ref[...] = jnp.zeros_like(s2_ref)
        c_ref[...] = jnp.zeros_like(c_ref)

    b = b_ref[0, 0, :]
    oh = (b[None, :] == lax.broadcasted_iota(jnp.int32, (NUM_GRAPHS, b.shape[0]), 0)
          ).astype(jnp.float32)
    s1_ref[...] += jnp.dot(oh, h1_ref[...], preferred_element_type=jnp.float32)
    s2_ref[...] += jnp.dot(oh, h2_ref[...], preferred_element_type=jnp.float32)
    c_ref[...] += jnp.sum(oh, axis=1)[:, None]


def _pool(batchs, h1, h2):
    B = 1000
    grid = N // B
    b3 = batchs.reshape(grid, 1, B)
    out = lambda: pl.BlockSpec((NUM_GRAPHS, D), lambda i: (0, 0))
    return pl.pallas_call(
        _pool_body,
        grid=(grid,),
        in_specs=[pl.BlockSpec((1, 1, B), lambda i: (i, 0, 0)),
                  pl.BlockSpec((B, D), lambda i: (i, 0)),
                  pl.BlockSpec((B, D), lambda i: (i, 0))],
        out_specs=[out(), out(), out()],
        out_shape=[jax.ShapeDtypeStruct((NUM_GRAPHS, D), jnp.float32)] * 3,
    )(b3, h1, h2)


def _head_body(s1_ref, s2_ref, c_ref, w1, b1, w2, b2, w3, b3, o_ref):
    c = jnp.maximum(c_ref[...], 1.0)
    xc = jnp.concatenate([s1_ref[...] / c, s2_ref[...] / c], axis=1)
    xc = jnp.maximum(jnp.dot(xc, w1[...], preferred_element_type=jnp.float32)
                     + b1[...], 0.0)
    xc = jnp.maximum(jnp.dot(xc, w2[...], preferred_element_type=jnp.float32)
                     + b2[...], 0.0)
    xc = jnp.dot(xc, w3[...], preferred_element_type=jnp.float32) + b3[...]
    m = jnp.max(xc, axis=1, keepdims=True)
    e = jnp.exp(xc - m)
    o_ref[...] = e / jnp.sum(e, axis=1, keepdims=True)


def _head(s1, s2, cnt, params):
    return pl.pallas_call(
        _head_body,
        out_shape=jax.ShapeDtypeStruct((NUM_GRAPHS, 2), jnp.float32),
    )(s1, s2, cnt,
      params["linl_W"], params["linl_b"].reshape(1, 2 * D),
      params["linl2_W"], params["linl2_b"].reshape(1, D),
      params["fc_W"], params["fc_b"].reshape(1, 2))


def _prep_edges(edge_index, edge_attr):
    pad = jnp.full((E_PAD - E,), N, jnp.int32)
    src = jnp.concatenate([edge_index[0], pad]).reshape(NS, CH, C)
    dst = jnp.concatenate([edge_index[1], pad]).reshape(NS, CH, C)
    ed = edge_attr.shape[1]
    ea = jnp.pad(edge_attr, ((0, E_PAD - E), (0, 16 - ed)))
    return src, dst, ea


def _padw(w):
    return jnp.pad(w, ((0, D - w.shape[0]), (0, D - w.shape[1])))


def _stack_params(plist, names, padder):
    return {n: jnp.stack([padder(p[n]) for p in plist]) for n in names}


def kernel(x, edge_index, edge_attr, batchs, x_apo, edge_index_apo,
           edge_attr_apo, params):
    src1, dst1, ea1 = _prep_edges(edge_index, edge_attr)
    src2, dst2, ea2 = _prep_edges(edge_index_apo, edge_attr_apo)

    x1 = jnp.pad(x, ((0, N_PAD - N), (0, D - x.shape[1])))
    x2 = jnp.pad(x_apo, ((0, N_PAD - N), (0, D - x_apo.shape[1])))

    def layer(h_in, src, dst, ea, p):
        q = _proj_one(h_in, _padw(p["Wq"]), p["bq"])
        k = _proj_one(h_in, _padw(p["Wk"]), p["bk"])
        v = _proj_one(h_in, _padw(p["Wv"]), p["bv"])
        s_pre = _proj_one(h_in, _padw(p["Ws"]), p["bs"])
        we16 = jnp.pad(p["We"], ((0, 16 - p["We"].shape[0]), (0, 0)))
        e = _eproj(ea, we16).reshape(NS, CH, C, D)
        po, den = _edge_kernel(q, k, v, src, dst, e)
        po2 = po.reshape(NC, OH, D)[:, :NH, :].reshape(N_PAD, D)
        return _epilogue(po2, den.reshape(NS, N_PAD), s_pre)

    h = layer(x1, src1, dst1, ea1, params["conv1"])
    for p in params["convs"]:
        h = layer(h, src1, dst1, ea1, p)
    h2 = layer(x2, src2, dst2, ea2, params["conv2"])
    for p in params["convs2"]:
        h2 = layer(h2, src2, dst2, ea2, p)

    s1, s2, cnt = _pool(batchs, h[:N], h2[:N])
    return _head(s1, s2, cnt, params)


# speedup vs baseline: 2.4806x; 1.0116x over previous
import math

import jax
import jax.numpy as jnp
from jax import lax
from jax.experimental import pallas as pl
from jax.experimental.pallas import tpu as pltpu
from jax.experimental.pallas import tpu_sc as plsc

N = 10000
E = 160000
D = 128
NUM_GRAPHS = 64

NC = 2
NS = 16
NW = NC * NS
L = 16

N_PAD = 10240
C = 128
CH = 80
E_PAD = NS * CH * C
NH = N_PAD // 2
OH = NH + 128
INV_SQRT_D = 1.0 / math.sqrt(float(D))


def _edge_body(q_hbm, k_hbm, v_hbm, src_hbm, dst_hbm, e_hbm,
               po_hbm, den_hbm,
               q_rows, k_rows, v_rows, e_rows,
               src_v, dst_v, dst_loc, den_tile, out_sh, sem, sem2):
    cid = lax.axis_index("c")
    sid = lax.axis_index("s")
    rows = OH // NS
    zero16 = jnp.zeros((L,), jnp.float32)

    def zden(i, carry):
        den_tile[pl.ds(i * L, L)] = zero16
        return carry

    lax.fori_loop(0, N_PAD // L, zden, None)

    def zrows(i, carry):
        for j in range(D // L):
            v_rows[i, pl.ds(j * L, L)] = zero16
        return carry

    lax.fori_loop(0, C, zrows, None)
    zbase = sid * rows
    for t in range(rows // C):
        pltpu.sync_copy(v_rows, out_sh.at[pl.ds(zbase + t * C, C)])
    rem = rows % C
    if rem:
        pltpu.sync_copy(v_rows.at[pl.ds(0, rem)],
                        out_sh.at[pl.ds(zbase + (rows // C) * C, rem)])
    plsc.subcore_barrier()
    half_base = cid * NH

    lane = lax.iota(jnp.int32, L)

    def chunk(c, carry):
        d_src = pltpu.async_copy(src_hbm.at[sid, c], src_v, sem2)
        d_dst = pltpu.async_copy(dst_hbm.at[sid, c], dst_v, sem2)
        d_e = pltpu.async_copy(e_hbm.at[sid, c], e_rows, sem2)
        d_src.wait()
        d_dst.wait()
        d_q = pltpu.async_copy(q_hbm.at[dst_v], q_rows, sem)
        d_k = pltpu.async_copy(k_hbm.at[src_v], k_rows, sem)
        d_v = pltpu.async_copy(v_hbm.at[src_v], v_rows, sem)
        d_e.wait()
        d_q.wait()
        d_k.wait()
        d_v.wait()

        def dot16(b, inner):
            base = b * L
            parts = []
            for l in range(L):
                i = base + l
                acc0 = jnp.zeros((L,), jnp.float32)
                acc1 = jnp.zeros((L,), jnp.float32)
                for j in range(0, D // L, 2):
                    sl0 = pl.ds(j * L, L)
                    sl1 = pl.ds((j + 1) * L, L)
                    acc0 = acc0 + q_rows[i, sl0] * (k_rows[i, sl0] + e_rows[i, sl0])
                    acc1 = acc1 + q_rows[i, sl1] * (k_rows[i, sl1] + e_rows[i, sl1])
                parts.append(jnp.where(lane == l, jnp.sum(acc0 + acc1), 0.0))
            while len(parts) > 1:
                parts = [parts[m] + parts[m + 1] for m in range(0, len(parts), 2)]
            alpha16 = parts[0]
            ex16 = jnp.exp(alpha16 * INV_SQRT_D)
            d16 = dst_v[pl.ds(base, L)]

            @pl.when(cid == 0)
            def _():
                plsc.addupdate_scatter(den_tile, [d16], ex16)

            dl = d16 - half_base
            dl = jnp.where((dl >= 0) & (dl < NH), dl, NH)
            dst_loc[pl.ds(base, L)] = dl
            for l in range(L):
                i = base + l
                ex = ex16[l]
                for j in range(D // L):
                    sl = pl.ds(j * L, L)
                    v_rows[i, sl] = (v_rows[i, sl] + e_rows[i, sl]) * ex
            return inner

        lax.fori_loop(0, C // L, dot16, None)

        pltpu.sync_copy(v_rows, out_sh.at[dst_loc], add=True)
        return carry

    lax.fori_loop(0, CH, chunk, None)

    @pl.when(cid == 0)
    def _():
        pltpu.sync_copy(den_tile, den_hbm.at[pl.ds(sid * N_PAD, N_PAD)])

    plsc.subcore_barrier()

    pltpu.sync_copy(out_sh.at[pl.ds(sid * rows, rows)],
                    po_hbm.at[pl.ds(cid * OH + sid * rows, rows)])


_edge_kernel = pl.kernel(
    _edge_body,
    out_type=(
        jax.ShapeDtypeStruct((NC * OH, D), jnp.float32),
        jax.ShapeDtypeStruct((NS * N_PAD,), jnp.float32),
    ),
    mesh=plsc.VectorSubcoreMesh(core_axis_name="c", subcore_axis_name="s",
                                num_cores=NC, num_subcores=NS),
    compiler_params=pltpu.CompilerParams(needs_layout_passes=False),
    scratch_types=[
        pltpu.VMEM((C, D), jnp.float32),
        pltpu.VMEM((C, D), jnp.float32),
        pltpu.VMEM((C, D), jnp.float32),
        pltpu.VMEM((C, D), jnp.float32),
        pltpu.VMEM((C,), jnp.int32),
        pltpu.VMEM((C,), jnp.int32),
        pltpu.VMEM((C,), jnp.int32),
        pltpu.VMEM((N_PAD,), jnp.float32),
        pltpu.VMEM_SHARED((OH, D), jnp.float32),
        pltpu.SemaphoreType.DMA,
        pltpu.SemaphoreType.DMA,
    ],
)


def _proj_body(x_ref, w_ref, b_ref, o_ref):
    o_ref[...] = (jnp.dot(x_ref[...], w_ref[...],
                          preferred_element_type=jnp.float32) + b_ref[...])


def _proj_one(x_pad, w, b):
    B = 2048
    grid = N_PAD // B
    return pl.pallas_call(
        _proj_body,
        grid=(grid,),
        in_specs=[pl.BlockSpec((B, D), lambda i: (i, 0)),
                  pl.BlockSpec((D, D), lambda i: (0, 0)),
                  pl.BlockSpec((1, D), lambda i: (0, 0))],
        out_specs=pl.BlockSpec((B, D), lambda i: (i, 0)),
        out_shape=jax.ShapeDtypeStruct((N_PAD, D), jnp.float32),
    )(x_pad, w, b.reshape(1, D))


def _eproj_body(ea_ref, we_ref, e_ref):
    e_ref[...] = jnp.dot(ea_ref[...], we_ref[...],
                         preferred_element_type=jnp.float32)


def _eproj(ea_pad, we16):
    B = 2048
    grid = E_PAD // B
    return pl.pallas_call(
        _eproj_body,
        grid=(grid,),
        in_specs=[pl.BlockSpec((B, 16), lambda i: (i, 0)),
                  pl.BlockSpec((16, D), lambda i: (0, 0))],
        out_specs=pl.BlockSpec((B, D), lambda i: (i, 0)),
        out_shape=jax.ShapeDtypeStruct((E_PAD, D), jnp.float32),
    )(ea_pad, we16)


def _epilogue_body(po_ref, den_ref, s_ref, h_ref):
    den = jnp.sum(den_ref[...], axis=0)[:, None]
    h_ref[...] = jnp.maximum(po_ref[...] / (den + 1e-16) + s_ref[...], 0.0)


def _epilogue(po, den, s_pre):
    B = 1280
    grid = N_PAD // B
    return pl.pallas_call(
        _epilogue_body,
        grid=(grid,),
        in_specs=[pl.BlockSpec((B, D), lambda i: (i, 0)),
                  pl.BlockSpec((NS, B), lambda i: (0, i)),
                  pl.BlockSpec((B, D), lambda i: (i, 0))],
        out_specs=pl.BlockSpec((B, D), lambda i: (i, 0)),
        out_shape=jax.ShapeDtypeStruct((N_PAD, D), jnp.float32),
    )(po, den, s_pre)


def _pool_body(b_ref, h1_ref, h2_ref, s1_ref, s2_ref, c_ref):
    @pl.when(pl.program_id(0) == 0)
    def _():
        s1_ref[...] = jnp.zeros_like(s1_ref)
        s2_ref[...] = jnp.zeros_like(s2_ref)
        c_ref[...] = jnp.zeros_like(c_ref)

    b = b_ref[0, 0, :]
    oh = (b[None, :] == lax.broadcasted_iota(jnp.int32, (NUM_GRAPHS, b.shape[0]), 0)
          ).astype(jnp.float32)
    s1_ref[...] += jnp.dot(oh, h1_ref[...], preferred_element_type=jnp.float32)
    s2_ref[...] += jnp.dot(oh, h2_ref[...], preferred_element_type=jnp.float32)
    c_ref[...] += jnp.sum(oh, axis=1)[:, None]


def _pool(batchs, h1, h2):
    B = 1000
    grid = N // B
    b3 = batchs.reshape(grid, 1, B)
    out = lambda: pl.BlockSpec((NUM_GRAPHS, D), lambda i: (0, 0))
    return pl.pallas_call(
        _pool_body,
        grid=(grid,),
        in_specs=[pl.BlockSpec((1, 1, B), lambda i: (i, 0, 0)),
                  pl.BlockSpec((B, D), lambda i: (i, 0)),
                  pl.BlockSpec((B, D), lambda i: (i, 0))],
        out_specs=[out(), out(), out()],
        out_shape=[jax.ShapeDtypeStruct((NUM_GRAPHS, D), jnp.float32)] * 3,
    )(b3, h1, h2)


def _head_body(s1_ref, s2_ref, c_ref, w1, b1, w2, b2, w3, b3, o_ref):
    c = jnp.maximum(c_ref[...], 1.0)
    xc = jnp.concatenate([s1_ref[...] / c, s2_ref[...] / c], axis=1)
    xc = jnp.maximum(jnp.dot(xc, w1[...], preferred_element_type=jnp.float32)
                     + b1[...], 0.0)
    xc = jnp.maximum(jnp.dot(xc, w2[...], preferred_element_type=jnp.float32)
                     + b2[...], 0.0)
    xc = jnp.dot(xc, w3[...], preferred_element_type=jnp.float32) + b3[...]
    m = jnp.max(xc, axis=1, keepdims=True)
    e = jnp.exp(xc - m)
    o_ref[...] = e / jnp.sum(e, axis=1, keepdims=True)


def _head(s1, s2, cnt, params):
    return pl.pallas_call(
        _head_body,
        out_shape=jax.ShapeDtypeStruct((NUM_GRAPHS, 2), jnp.float32),
    )(s1, s2, cnt,
      params["linl_W"], params["linl_b"].reshape(1, 2 * D),
      params["linl2_W"], params["linl2_b"].reshape(1, D),
      params["fc_W"], params["fc_b"].reshape(1, 2))


def _prep_edges(edge_index, edge_attr):
    pad = jnp.full((E_PAD - E,), N, jnp.int32)
    src = jnp.concatenate([edge_index[0], pad]).reshape(NS, CH, C)
    dst = jnp.concatenate([edge_index[1], pad]).reshape(NS, CH, C)
    ed = edge_attr.shape[1]
    ea = jnp.pad(edge_attr, ((0, E_PAD - E), (0, 16 - ed)))
    return src, dst, ea


def _padw(w):
    return jnp.pad(w, ((0, D - w.shape[0]), (0, D - w.shape[1])))


def _stack_params(plist, names, padder):
    return {n: jnp.stack([padder(p[n]) for p in plist]) for n in names}


def kernel(x, edge_index, edge_attr, batchs, x_apo, edge_index_apo,
           edge_attr_apo, params):
    src1, dst1, ea1 = _prep_edges(edge_index, edge_attr)
    src2, dst2, ea2 = _prep_edges(edge_index_apo, edge_attr_apo)

    x1 = jnp.pad(x, ((0, N_PAD - N), (0, D - x.shape[1])))
    x2 = jnp.pad(x_apo, ((0, N_PAD - N), (0, D - x_apo.shape[1])))

    def layer(h_in, src, dst, ea, p):
        q = _proj_one(h_in, _padw(p["Wq"]), p["bq"])
        k = _proj_one(h_in, _padw(p["Wk"]), p["bk"])
        v = _proj_one(h_in, _padw(p["Wv"]), p["bv"])
        s_pre = _proj_one(h_in, _padw(p["Ws"]), p["bs"])
        we16 = jnp.pad(p["We"], ((0, 16 - p["We"].shape[0]), (0, 0)))
        e = _eproj(ea, we16).reshape(NS, CH, C, D)
        po, den = _edge_kernel(q, k, v, src, dst, e)
        po2 = po.reshape(NC, OH, D)[:, :NH, :].reshape(N_PAD, D)
        return _epilogue(po2, den.reshape(NS, N_PAD), s_pre)

    h = layer(x1, src1, dst1, ea1, params["conv1"])
    for p in params["convs"]:
        h = layer(h, src1, dst1, ea1, p)
    h2 = layer(x2, src2, dst2, ea2, params["conv2"])
    for p in params["convs2"]:
        h2 = layer(h2, src2, dst2, ea2, p)

    s1, s2, cnt = _pool(batchs, h[:N], h2[:N])
    return _head(s1, s2, cnt, params)


# speedup vs baseline: 5.2218x; 2.1050x over previous
import math

import jax
import jax.numpy as jnp
from jax import lax
from jax.experimental import pallas as pl
from jax.experimental.pallas import tpu as pltpu
from jax.experimental.pallas import tpu_sc as plsc

N = 10000
E = 160000
D = 128
NUM_GRAPHS = 64

NC = 2
NS = 16
NW = NC * NS
L = 16

N_PAD = 10240
C = 128
CH = 80
E_PAD = NS * CH * C
CHG = E_PAD // (NW * C)
INV_SQRT_D = 1.0 / math.sqrt(float(D))


def _gather_body(q_hbm, k_hbm, v_hbm, src_hbm, dst_hbm,
                 qd_hbm, kd_hbm, vd_hbm,
                 q0, k0, v0, q1, k1, v1, src_v, dst0, dst1, semA, semB):
    cid = lax.axis_index("c")
    sid = lax.axis_index("s")
    wid = cid * NS + sid

    def issue(c, qb, kb, vb, dstb, sem):
        pltpu.sync_copy(src_hbm.at[wid, c], src_v)
        pltpu.sync_copy(dst_hbm.at[wid, c], dstb)
        pltpu.async_copy(q_hbm.at[dstb], qb, sem)
        pltpu.async_copy(k_hbm.at[src_v], kb, sem)
        pltpu.async_copy(v_hbm.at[src_v], vb, sem)

    def flush(c, qb, kb, vb, dstb, sem):
        pltpu.make_async_copy(q_hbm.at[dstb], qb, sem).wait()
        pltpu.make_async_copy(k_hbm.at[src_v], kb, sem).wait()
        pltpu.make_async_copy(v_hbm.at[src_v], vb, sem).wait()
        base = (wid * CHG + c) * C
        pltpu.sync_copy(qb, qd_hbm.at[pl.ds(base, C)])
        pltpu.sync_copy(kb, kd_hbm.at[pl.ds(base, C)])
        pltpu.sync_copy(vb, vd_hbm.at[pl.ds(base, C)])

    issue(0, q0, k0, v0, dst0, semA)
    issue(1, q1, k1, v1, dst1, semB)

    def pair(p, carry):
        c0 = p * 2
        flush(c0, q0, k0, v0, dst0, semA)

        @pl.when(p + 1 < CHG // 2)
        def _():
            issue(c0 + 2, q0, k0, v0, dst0, semA)

        flush(c0 + 1, q1, k1, v1, dst1, semB)

        @pl.when(p + 1 < CHG // 2)
        def _():
            issue(c0 + 3, q1, k1, v1, dst1, semB)

        return carry

    lax.fori_loop(0, CHG // 2, pair, None)


_gather_kernel = pl.kernel(
    _gather_body,
    out_type=(
        jax.ShapeDtypeStruct((E_PAD, D), jnp.float32),
        jax.ShapeDtypeStruct((E_PAD, D), jnp.float32),
        jax.ShapeDtypeStruct((E_PAD, D), jnp.float32),
    ),
    mesh=plsc.VectorSubcoreMesh(core_axis_name="c", subcore_axis_name="s",
                                num_cores=NC, num_subcores=NS),
    compiler_params=pltpu.CompilerParams(needs_layout_passes=False),
    scratch_types=[
        pltpu.VMEM((C, D), jnp.float32),
        pltpu.VMEM((C, D), jnp.float32),
        pltpu.VMEM((C, D), jnp.float32),
        pltpu.VMEM((C, D), jnp.float32),
        pltpu.VMEM((C, D), jnp.float32),
        pltpu.VMEM((C, D), jnp.float32),
        pltpu.VMEM((C,), jnp.int32),
        pltpu.VMEM((C,), jnp.int32),
        pltpu.VMEM((C,), jnp.int32),
        pltpu.SemaphoreType.DMA,
        pltpu.SemaphoreType.DMA,
    ],
)


def _scatter_body(msg_hbm, ex_hbm, dst_hbm, po_hbm, den_hbm,
                  m0, m1, ex_v, dst0, dst1, den_tile, out_sh, semA, semB):
    cid = lax.axis_index("c")
    sid = lax.axis_index("s")
    wid = cid * NS + sid
    rows = N_PAD // NS
    zero16 = jnp.zeros((L,), jnp.float32)

    def zden(i, carry):
        den_tile[pl.ds(i * L, L)] = zero16
        return carry

    lax.fori_loop(0, N_PAD // L, zden, None)

    def zrows(i, carry):
        for j in range(D // L):
            m0[i, pl.ds(j * L, L)] = zero16
        return carry

    lax.fori_loop(0, C, zrows, None)
    zbase = sid * rows
    for t in range(rows // C):
        pltpu.sync_copy(m0, out_sh.at[pl.ds(zbase + t * C, C)])
    rem = rows % C
    if rem:
        pltpu.sync_copy(m0.at[pl.ds(0, rem)],
                        out_sh.at[pl.ds(zbase + (rows // C) * C, rem)])
    plsc.subcore_barrier()

    def issue(c, mb, dstb, sem):
        pltpu.sync_copy(dst_hbm.at[wid, c], dstb)
        pltpu.async_copy(msg_hbm.at[pl.ds((wid * CHG + c) * C, C)], mb, sem)

    def drain(c, mb, dstb, sem):
        pltpu.make_async_copy(msg_hbm.at[pl.ds(0, C)], mb, sem).wait()
        pltpu.sync_copy(ex_hbm.at[wid, c], ex_v)

        def den16(b, carry):
            plsc.addupdate_scatter(den_tile, [dstb[pl.ds(b * L, L)]],
                                   ex_v[pl.ds(b * L, L)])
            return carry

        lax.fori_loop(0, C // L, den16, None)
        pltpu.sync_copy(mb, out_sh.at[dstb], add=True)

    issue(0, m0, dst0, semA)
    issue(1, m1, dst1, semB)

    def pair(p, carry):
        c0 = p * 2
        drain(c0, m0, dst0, semA)

        @pl.when(p + 1 < CHG // 2)
        def _():
            issue(c0 + 2, m0, dst0, semA)

        drain(c0 + 1, m1, dst1, semB)

        @pl.when(p + 1 < CHG // 2)
        def _():
            issue(c0 + 3, m1, dst1, semB)

        return carry

    lax.fori_loop(0, CHG // 2, pair, None)

    pltpu.sync_copy(den_tile, den_hbm.at[pl.ds(wid * N_PAD, N_PAD)])
    plsc.subcore_barrier()

    pltpu.sync_copy(out_sh.at[pl.ds(sid * rows, rows)],
                    po_hbm.at[pl.ds(cid * N_PAD + sid * rows, rows)])


_scatter_kernel = pl.kernel(
    _scatter_body,
    out_type=(
        jax.ShapeDtypeStruct((NC * N_PAD, D), jnp.float32),
        jax.ShapeDtypeStruct((NW * N_PAD,), jnp.float32),
    ),
    mesh=plsc.VectorSubcoreMesh(core_axis_name="c", subcore_axis_name="s",
                                num_cores=NC, num_subcores=NS),
    compiler_params=pltpu.CompilerParams(needs_layout_passes=False),
    scratch_types=[
        pltpu.VMEM((C, D), jnp.float32),
        pltpu.VMEM((C, D), jnp.float32),
        pltpu.VMEM((C,), jnp.float32),
        pltpu.VMEM((C,), jnp.int32),
        pltpu.VMEM((C,), jnp.int32),
        pltpu.VMEM((N_PAD,), jnp.float32),
        pltpu.VMEM_SHARED((N_PAD, D), jnp.float32),
        pltpu.SemaphoreType.DMA,
        pltpu.SemaphoreType.DMA,
    ],
)


def _proj_body(x_ref, w_ref, b_ref, o_ref):
    o_ref[...] = (jnp.dot(x_ref[...], w_ref[...],
                          preferred_element_type=jnp.float32) + b_ref[...])


def _proj_one(x_pad, w, b):
    B = 2048
    grid = N_PAD // B
    return pl.pallas_call(
        _proj_body,
        grid=(grid,),
        in_specs=[pl.BlockSpec((B, D), lambda i: (i, 0)),
                  pl.BlockSpec((D, D), lambda i: (0, 0)),
                  pl.BlockSpec((1, D), lambda i: (0, 0))],
        out_specs=pl.BlockSpec((B, D), lambda i: (i, 0)),
        out_shape=jax.ShapeDtypeStruct((N_PAD, D), jnp.float32),
    )(x_pad, w, b.reshape(1, D))


def _eproj_body(ea_ref, we_ref, e_ref):
    e_ref[...] = jnp.dot(ea_ref[...], we_ref[...],
                         preferred_element_type=jnp.float32)


def _eproj(ea_pad, we16):
    B = 2048
    grid = E_PAD // B
    return pl.pallas_call(
        _eproj_body,
        grid=(grid,),
        in_specs=[pl.BlockSpec((B, 16), lambda i: (i, 0)),
                  pl.BlockSpec((16, D), lambda i: (0, 0))],
        out_specs=pl.BlockSpec((B, D), lambda i: (i, 0)),
        out_shape=jax.ShapeDtypeStruct((E_PAD, D), jnp.float32),
    )(ea_pad, we16)


def _attn_body(qd_ref, kd_ref, vd_ref, e_ref, msg_ref, ex_ref):
    qd = qd_ref[...]
    e = e_ref[...]
    alpha = jnp.sum(qd * (kd_ref[...] + e), axis=1) * INV_SQRT_D
    ex = jnp.exp(alpha)
    msg_ref[...] = ex[:, None] * (vd_ref[...] + e)
    ex_ref[...] = ex.reshape(ex_ref.shape)


def _attn(qd, kd, vd, e):
    B = 2048
    grid = E_PAD // B
    blk = pl.BlockSpec((B, D), lambda i: (i, 0))
    return pl.pallas_call(
        _attn_body,
        grid=(grid,),
        in_specs=[blk, blk, blk, blk],
        out_specs=[blk, pl.BlockSpec((B // 128, 128), lambda i: (i, 0))],
        out_shape=[jax.ShapeDtypeStruct((E_PAD, D), jnp.float32),
                   jax.ShapeDtypeStruct((E_PAD // 128, 128), jnp.float32)],
    )(qd, kd, vd, e)


def _epilogue_body(po_ref, den_ref, s_ref, h_ref):
    msg = po_ref[0] + po_ref[1]
    den = jnp.sum(den_ref[...], axis=0)[:, None]
    h_ref[...] = jnp.maximum(msg / (den + 1e-16) + s_ref[...], 0.0)


def _epilogue(po, den, s_pre):
    B = 1280
    grid = N_PAD // B
    return pl.pallas_call(
        _epilogue_body,
        grid=(grid,),
        in_specs=[pl.BlockSpec((NC, B, D), lambda i: (0, i, 0)),
                  pl.BlockSpec((NW, B), lambda i: (0, i)),
                  pl.BlockSpec((B, D), lambda i: (i, 0))],
        out_specs=pl.BlockSpec((B, D), lambda i: (i, 0)),
        out_shape=jax.ShapeDtypeStruct((N_PAD, D), jnp.float32),
    )(po, den, s_pre)


def _pool_body(b_ref, h1_ref, h2_ref, s1_ref, s2_ref, c_ref):
    @pl.when(pl.program_id(0) == 0)
    def _():
        s1_ref[...] = jnp.zeros_like(s1_ref)
        s2_ref[...] = jnp.zeros_like(s2_ref)
        c_ref[...] = jnp.zeros_like(c_ref)

    b = b_ref[0, 0, :]
    oh = (b[None, :] == lax.broadcasted_iota(jnp.int32, (NUM_GRAPHS, b.shape[0]), 0)
          ).astype(jnp.float32)
    s1_ref[...] += jnp.dot(oh, h1_ref[...], preferred_element_type=jnp.float32)
    s2_ref[...] += jnp.dot(oh, h2_ref[...], preferred_element_type=jnp.float32)
    c_ref[...] += jnp.sum(oh, axis=1)[:, None]


def _pool(batchs, h1, h2):
    B = 1000
    grid = N // B
    b3 = batchs.reshape(grid, 1, B)
    out = lambda: pl.BlockSpec((NUM_GRAPHS, D), lambda i: (0, 0))
    return pl.pallas_call(
        _pool_body,
        grid=(grid,),
        in_specs=[pl.BlockSpec((1, 1, B), lambda i: (i, 0, 0)),
                  pl.BlockSpec((B, D), lambda i: (i, 0)),
                  pl.BlockSpec((B, D), lambda i: (i, 0))],
        out_specs=[out(), out(), out()],
        out_shape=[jax.ShapeDtypeStruct((NUM_GRAPHS, D), jnp.float32)] * 3,
    )(b3, h1, h2)


def _head_body(s1_ref, s2_ref, c_ref, w1, b1, w2, b2, w3, b3, o_ref):
    c = jnp.maximum(c_ref[...], 1.0)
    xc = jnp.concatenate([s1_ref[...] / c, s2_ref[...] / c], axis=1)
    xc = jnp.maximum(jnp.dot(xc, w1[...], preferred_element_type=jnp.float32)
                     + b1[...], 0.0)
    xc = jnp.maximum(jnp.dot(xc, w2[...], preferred_element_type=jnp.float32)
                     + b2[...], 0.0)
    xc = jnp.dot(xc, w3[...], preferred_element_type=jnp.float32) + b3[...]
    m = jnp.max(xc, axis=1, keepdims=True)
    e = jnp.exp(xc - m)
    o_ref[...] = e / jnp.sum(e, axis=1, keepdims=True)


def _head(s1, s2, cnt, params):
    return pl.pallas_call(
        _head_body,
        out_shape=jax.ShapeDtypeStruct((NUM_GRAPHS, 2), jnp.float32),
    )(s1, s2, cnt,
      params["linl_W"], params["linl_b"].reshape(1, 2 * D),
      params["linl2_W"], params["linl2_b"].reshape(1, D),
      params["fc_W"], params["fc_b"].reshape(1, 2))


def _prep_edges(edge_index, edge_attr):
    pad = jnp.full((E_PAD - E,), N, jnp.int32)
    src = jnp.concatenate([edge_index[0], pad]).reshape(NW, CHG, C)
    dst = jnp.concatenate([edge_index[1], pad]).reshape(NW, CHG, C)
    ed = edge_attr.shape[1]
    ea = jnp.pad(edge_attr, ((0, E_PAD - E), (0, 16 - ed)))
    return src, dst, ea


def _padw(w):
    return jnp.pad(w, ((0, D - w.shape[0]), (0, D - w.shape[1])))


def _stack_params(plist, names, padder):
    return {n: jnp.stack([padder(p[n]) for p in plist]) for n in names}


def kernel(x, edge_index, edge_attr, batchs, x_apo, edge_index_apo,
           edge_attr_apo, params):
    src1, dst1, ea1 = _prep_edges(edge_index, edge_attr)
    src2, dst2, ea2 = _prep_edges(edge_index_apo, edge_attr_apo)

    x1 = jnp.pad(x, ((0, N_PAD - N), (0, D - x.shape[1])))
    x2 = jnp.pad(x_apo, ((0, N_PAD - N), (0, D - x_apo.shape[1])))

    def layer(h_in, src, dst, ea, p):
        q = _proj_one(h_in, _padw(p["Wq"]), p["bq"])
        k = _proj_one(h_in, _padw(p["Wk"]), p["bk"])
        v = _proj_one(h_in, _padw(p["Wv"]), p["bv"])
        s_pre = _proj_one(h_in, _padw(p["Ws"]), p["bs"])
        we16 = jnp.pad(p["We"], ((0, 16 - p["We"].shape[0]), (0, 0)))
        e = _eproj(ea, we16)
        qd, kd, vd = _gather_kernel(q, k, v, src, dst)
        msg, ex = _attn(qd, kd, vd, e)
        po, den = _scatter_kernel(msg, ex.reshape(NW, CHG, C), dst)
        return _epilogue(po.reshape(NC, N_PAD, D), den.reshape(NW, N_PAD),
                         s_pre)

    h = layer(x1, src1, dst1, ea1, params["conv1"])
    for p in params["convs"]:
        h = layer(h, src1, dst1, ea1, p)
    h2 = layer(x2, src2, dst2, ea2, params["conv2"])
    for p in params["convs2"]:
        h2 = layer(h2, src2, dst2, ea2, p)

    s1, s2, cnt = _pool(batchs, h[:N], h2[:N])
    return _head(s1, s2, cnt, params)


# speedup vs baseline: 5.2251x; 1.0006x over previous
import math

import jax
import jax.numpy as jnp
from jax import lax
from jax.experimental import pallas as pl
from jax.experimental.pallas import tpu as pltpu
from jax.experimental.pallas import tpu_sc as plsc

N = 10000
E = 160000
D = 128
NUM_GRAPHS = 64

NC = 2
NS = 16
NW = NC * NS
L = 16

N_PAD = 10240
C = 128
CH = 80
E_PAD = NS * CH * C
CHG = E_PAD // (NW * C)
INV_SQRT_D = 1.0 / math.sqrt(float(D))


def _gather_body(q_hbm, k_hbm, v_hbm, src_hbm, dst_hbm,
                 qd_hbm, kd_hbm, vd_hbm,
                 q0, k0, v0, q1, k1, v1, src_v, dst0, dst1, semA, semB):
    cid = lax.axis_index("c")
    sid = lax.axis_index("s")
    wid = cid * NS + sid

    def issue(c, qb, kb, vb, dstb, sem):
        pltpu.sync_copy(src_hbm.at[wid, c], src_v)
        pltpu.sync_copy(dst_hbm.at[wid, c], dstb)
        pltpu.async_copy(q_hbm.at[dstb], qb, sem)
        pltpu.async_copy(k_hbm.at[src_v], kb, sem)
        pltpu.async_copy(v_hbm.at[src_v], vb, sem)

    def flush(c, qb, kb, vb, dstb, sem):
        pltpu.make_async_copy(q_hbm.at[dstb], qb, sem).wait()
        pltpu.make_async_copy(k_hbm.at[src_v], kb, sem).wait()
        pltpu.make_async_copy(v_hbm.at[src_v], vb, sem).wait()
        base = (wid * CHG + c) * C
        pltpu.sync_copy(qb, qd_hbm.at[pl.ds(base, C)])
        pltpu.sync_copy(kb, kd_hbm.at[pl.ds(base, C)])
        pltpu.sync_copy(vb, vd_hbm.at[pl.ds(base, C)])

    issue(0, q0, k0, v0, dst0, semA)
    issue(1, q1, k1, v1, dst1, semB)

    def pair(p, carry):
        c0 = p * 2
        flush(c0, q0, k0, v0, dst0, semA)

        @pl.when(p + 1 < CHG // 2)
        def _():
            issue(c0 + 2, q0, k0, v0, dst0, semA)

        flush(c0 + 1, q1, k1, v1, dst1, semB)

        @pl.when(p + 1 < CHG // 2)
        def _():
            issue(c0 + 3, q1, k1, v1, dst1, semB)

        return carry

    lax.fori_loop(0, CHG // 2, pair, None)


_gather_kernel = pl.kernel(
    _gather_body,
    out_type=(
        jax.ShapeDtypeStruct((E_PAD, D), jnp.float32),
        jax.ShapeDtypeStruct((E_PAD, D), jnp.float32),
        jax.ShapeDtypeStruct((E_PAD, D), jnp.float32),
    ),
    mesh=plsc.VectorSubcoreMesh(core_axis_name="c", subcore_axis_name="s",
                                num_cores=NC, num_subcores=NS),
    compiler_params=pltpu.CompilerParams(needs_layout_passes=False),
    scratch_types=[
        pltpu.VMEM((C, D), jnp.float32),
        pltpu.VMEM((C, D), jnp.float32),
        pltpu.VMEM((C, D), jnp.float32),
        pltpu.VMEM((C, D), jnp.float32),
        pltpu.VMEM((C, D), jnp.float32),
        pltpu.VMEM((C, D), jnp.float32),
        pltpu.VMEM((C,), jnp.int32),
        pltpu.VMEM((C,), jnp.int32),
        pltpu.VMEM((C,), jnp.int32),
        pltpu.SemaphoreType.DMA,
        pltpu.SemaphoreType.DMA,
    ],
)


def _scatter_body(msg_hbm, ex_hbm, dst_hbm, po_hbm, den_hbm,
                  m0, m1, ex_v, dst0, dst1, den_tile, out_sh,
                  semA, semB, semS0, semS1):
    cid = lax.axis_index("c")
    sid = lax.axis_index("s")
    wid = cid * NS + sid
    rows = N_PAD // NS
    zero16 = jnp.zeros((L,), jnp.float32)

    def zden(i, carry):
        den_tile[pl.ds(i * L, L)] = zero16
        return carry

    lax.fori_loop(0, N_PAD // L, zden, None)

    def zrows(i, carry):
        for j in range(D // L):
            m0[i, pl.ds(j * L, L)] = zero16
        return carry

    lax.fori_loop(0, C, zrows, None)
    zbase = sid * rows
    for t in range(rows // C):
        pltpu.sync_copy(m0, out_sh.at[pl.ds(zbase + t * C, C)])
    rem = rows % C
    if rem:
        pltpu.sync_copy(m0.at[pl.ds(0, rem)],
                        out_sh.at[pl.ds(zbase + (rows // C) * C, rem)])
    plsc.subcore_barrier()

    def issue(c, mb, dstb, sem):
        pltpu.sync_copy(dst_hbm.at[wid, c], dstb)
        pltpu.async_copy(msg_hbm.at[pl.ds((wid * CHG + c) * C, C)], mb, sem)

    def drain(c, mb, dstb, sem, semS):
        pltpu.make_async_copy(msg_hbm.at[pl.ds(0, C)], mb, sem).wait()
        pltpu.sync_copy(ex_hbm.at[wid, c], ex_v)

        def den16(b, carry):
            plsc.addupdate_scatter(den_tile, [dstb[pl.ds(b * L, L)]],
                                   ex_v[pl.ds(b * L, L)])
            return carry

        lax.fori_loop(0, C // L, den16, None)
        pltpu.async_copy(mb, out_sh.at[dstb], semS, add=True)

    def scatter_wait(mb, dstb, semS):
        pltpu.make_async_copy(mb, out_sh.at[dstb], semS).wait()

    issue(0, m0, dst0, semA)
    issue(1, m1, dst1, semB)

    def pair(p, carry):
        c0 = p * 2
        drain(c0, m0, dst0, semA, semS0)

        @pl.when(p + 1 < CHG // 2)
        def _():
            scatter_wait(m0, dst0, semS0)
            issue(c0 + 2, m0, dst0, semA)

        drain(c0 + 1, m1, dst1, semB, semS1)

        @pl.when(p + 1 < CHG // 2)
        def _():
            scatter_wait(m1, dst1, semS1)
            issue(c0 + 3, m1, dst1, semB)

        return carry

    lax.fori_loop(0, CHG // 2, pair, None)
    scatter_wait(m0, dst0, semS0)
    scatter_wait(m1, dst1, semS1)

    pltpu.sync_copy(den_tile, den_hbm.at[pl.ds(wid * N_PAD, N_PAD)])
    plsc.subcore_barrier()

    pltpu.sync_copy(out_sh.at[pl.ds(sid * rows, rows)],
                    po_hbm.at[pl.ds(cid * N_PAD + sid * rows, rows)])


_scatter_kernel = pl.kernel(
    _scatter_body,
    out_type=(
        jax.ShapeDtypeStruct((NC * N_PAD, D), jnp.float32),
        jax.ShapeDtypeStruct((NW * N_PAD,), jnp.float32),
    ),
    mesh=plsc.VectorSubcoreMesh(core_axis_name="c", subcore_axis_name="s",
                                num_cores=NC, num_subcores=NS),
    compiler_params=pltpu.CompilerParams(needs_layout_passes=False),
    scratch_types=[
        pltpu.VMEM((C, D), jnp.float32),
        pltpu.VMEM((C, D), jnp.float32),
        pltpu.VMEM((C,), jnp.float32),
        pltpu.VMEM((C,), jnp.int32),
        pltpu.VMEM((C,), jnp.int32),
        pltpu.VMEM((N_PAD,), jnp.float32),
        pltpu.VMEM_SHARED((N_PAD, D), jnp.float32),
        pltpu.SemaphoreType.DMA,
        pltpu.SemaphoreType.DMA,
        pltpu.SemaphoreType.DMA,
        pltpu.SemaphoreType.DMA,
    ],
)


def _proj_body(x_ref, w_ref, b_ref, o_ref):
    o_ref[...] = (jnp.dot(x_ref[...], w_ref[...],
                          preferred_element_type=jnp.float32) + b_ref[...])


def _proj_one(x_pad, w, b):
    B = 2048
    grid = N_PAD // B
    return pl.pallas_call(
        _proj_body,
        grid=(grid,),
        in_specs=[pl.BlockSpec((B, D), lambda i: (i, 0)),
                  pl.BlockSpec((D, D), lambda i: (0, 0)),
                  pl.BlockSpec((1, D), lambda i: (0, 0))],
        out_specs=pl.BlockSpec((B, D), lambda i: (i, 0)),
        out_shape=jax.ShapeDtypeStruct((N_PAD, D), jnp.float32),
    )(x_pad, w, b.reshape(1, D))


def _eproj_body(ea_ref, we_ref, e_ref):
    e_ref[...] = jnp.dot(ea_ref[...], we_ref[...],
                         preferred_element_type=jnp.float32)


def _eproj(ea_pad, we16):
    B = 2048
    grid = E_PAD // B
    return pl.pallas_call(
        _eproj_body,
        grid=(grid,),
        in_specs=[pl.BlockSpec((B, 16), lambda i: (i, 0)),
                  pl.BlockSpec((16, D), lambda i: (0, 0))],
        out_specs=pl.BlockSpec((B, D), lambda i: (i, 0)),
        out_shape=jax.ShapeDtypeStruct((E_PAD, D), jnp.float32),
    )(ea_pad, we16)


def _attn_body(qd_ref, kd_ref, vd_ref, e_ref, msg_ref, ex_ref):
    qd = qd_ref[...]
    e = e_ref[...]
    alpha = jnp.sum(qd * (kd_ref[...] + e), axis=1) * INV_SQRT_D
    ex = jnp.exp(alpha)
    msg_ref[...] = ex[:, None] * (vd_ref[...] + e)
    ex_ref[...] = ex.reshape(ex_ref.shape)


def _attn(qd, kd, vd, e):
    B = 2048
    grid = E_PAD // B
    blk = pl.BlockSpec((B, D), lambda i: (i, 0))
    return pl.pallas_call(
        _attn_body,
        grid=(grid,),
        in_specs=[blk, blk, blk, blk],
        out_specs=[blk, pl.BlockSpec((B // 128, 128), lambda i: (i, 0))],
        out_shape=[jax.ShapeDtypeStruct((E_PAD, D), jnp.float32),
                   jax.ShapeDtypeStruct((E_PAD // 128, 128), jnp.float32)],
    )(qd, kd, vd, e)


def _epilogue_body(po_ref, den_ref, s_ref, h_ref):
    msg = po_ref[0] + po_ref[1]
    den = jnp.sum(den_ref[...], axis=0)[:, None]
    h_ref[...] = jnp.maximum(msg / (den + 1e-16) + s_ref[...], 0.0)


def _epilogue(po, den, s_pre):
    B = 1280
    grid = N_PAD // B
    return pl.pallas_call(
        _epilogue_body,
        grid=(grid,),
        in_specs=[pl.BlockSpec((NC, B, D), lambda i: (0, i, 0)),
                  pl.BlockSpec((NW, B), lambda i: (0, i)),
                  pl.BlockSpec((B, D), lambda i: (i, 0))],
        out_specs=pl.BlockSpec((B, D), lambda i: (i, 0)),
        out_shape=jax.ShapeDtypeStruct((N_PAD, D), jnp.float32),
    )(po, den, s_pre)


def _pool_body(b_ref, h1_ref, h2_ref, s1_ref, s2_ref, c_ref):
    @pl.when(pl.program_id(0) == 0)
    def _():
        s1_ref[...] = jnp.zeros_like(s1_ref)
        s2_ref[...] = jnp.zeros_like(s2_ref)
        c_ref[...] = jnp.zeros_like(c_ref)

    b = b_ref[0, 0, :]
    oh = (b[None, :] == lax.broadcasted_iota(jnp.int32, (NUM_GRAPHS, b.shape[0]), 0)
          ).astype(jnp.float32)
    s1_ref[...] += jnp.dot(oh, h1_ref[...], preferred_element_type=jnp.float32)
    s2_ref[...] += jnp.dot(oh, h2_ref[...], preferred_element_type=jnp.float32)
    c_ref[...] += jnp.sum(oh, axis=1)[:, None]


def _pool(batchs, h1, h2):
    B = 1000
    grid = N // B
    b3 = batchs.reshape(grid, 1, B)
    out = lambda: pl.BlockSpec((NUM_GRAPHS, D), lambda i: (0, 0))
    return pl.pallas_call(
        _pool_body,
        grid=(grid,),
        in_specs=[pl.BlockSpec((1, 1, B), lambda i: (i, 0, 0)),
                  pl.BlockSpec((B, D), lambda i: (i, 0)),
                  pl.BlockSpec((B, D), lambda i: (i, 0))],
        out_specs=[out(), out(), out()],
        out_shape=[jax.ShapeDtypeStruct((NUM_GRAPHS, D), jnp.float32)] * 3,
    )(b3, h1, h2)


def _head_body(s1_ref, s2_ref, c_ref, w1, b1, w2, b2, w3, b3, o_ref):
    c = jnp.maximum(c_ref[...], 1.0)
    xc = jnp.concatenate([s1_ref[...] / c, s2_ref[...] / c], axis=1)
    xc = jnp.maximum(jnp.dot(xc, w1[...], preferred_element_type=jnp.float32)
                     + b1[...], 0.0)
    xc = jnp.maximum(jnp.dot(xc, w2[...], preferred_element_type=jnp.float32)
                     + b2[...], 0.0)
    xc = jnp.dot(xc, w3[...], preferred_element_type=jnp.float32) + b3[...]
    m = jnp.max(xc, axis=1, keepdims=True)
    e = jnp.exp(xc - m)
    o_ref[...] = e / jnp.sum(e, axis=1, keepdims=True)


def _head(s1, s2, cnt, params):
    return pl.pallas_call(
        _head_body,
        out_shape=jax.ShapeDtypeStruct((NUM_GRAPHS, 2), jnp.float32),
    )(s1, s2, cnt,
      params["linl_W"], params["linl_b"].reshape(1, 2 * D),
      params["linl2_W"], params["linl2_b"].reshape(1, D),
      params["fc_W"], params["fc_b"].reshape(1, 2))


def _prep_edges(edge_index, edge_attr):
    pad = jnp.full((E_PAD - E,), N, jnp.int32)
    src = jnp.concatenate([edge_index[0], pad]).reshape(NW, CHG, C)
    dst = jnp.concatenate([edge_index[1], pad]).reshape(NW, CHG, C)
    ed = edge_attr.shape[1]
    ea = jnp.pad(edge_attr, ((0, E_PAD - E), (0, 16 - ed)))
    return src, dst, ea


def _padw(w):
    return jnp.pad(w, ((0, D - w.shape[0]), (0, D - w.shape[1])))


def _stack_params(plist, names, padder):
    return {n: jnp.stack([padder(p[n]) for p in plist]) for n in names}


def kernel(x, edge_index, edge_attr, batchs, x_apo, edge_index_apo,
           edge_attr_apo, params):
    src1, dst1, ea1 = _prep_edges(edge_index, edge_attr)
    src2, dst2, ea2 = _prep_edges(edge_index_apo, edge_attr_apo)

    x1 = jnp.pad(x, ((0, N_PAD - N), (0, D - x.shape[1])))
    x2 = jnp.pad(x_apo, ((0, N_PAD - N), (0, D - x_apo.shape[1])))

    def layer(h_in, src, dst, ea, p):
        q = _proj_one(h_in, _padw(p["Wq"]), p["bq"])
        k = _proj_one(h_in, _padw(p["Wk"]), p["bk"])
        v = _proj_one(h_in, _padw(p["Wv"]), p["bv"])
        s_pre = _proj_one(h_in, _padw(p["Ws"]), p["bs"])
        we16 = jnp.pad(p["We"], ((0, 16 - p["We"].shape[0]), (0, 0)))
        e = _eproj(ea, we16)
        qd, kd, vd = _gather_kernel(q, k, v, src, dst)
        msg, ex = _attn(qd, kd, vd, e)
        po, den = _scatter_kernel(msg, ex.reshape(NW, CHG, C), dst)
        return _epilogue(po.reshape(NC, N_PAD, D), den.reshape(NW, N_PAD),
                         s_pre)

    h = layer(x1, src1, dst1, ea1, params["conv1"])
    for p in params["convs"]:
        h = layer(h, src1, dst1, ea1, p)
    h2 = layer(x2, src2, dst2, ea2, params["conv2"])
    for p in params["convs2"]:
        h2 = layer(h2, src2, dst2, ea2, p)

    s1, s2, cnt = _pool(batchs, h[:N], h2[:N])
    return _head(s1, s2, cnt, params)
